# baseline (speedup 1.0000x reference)
import jax
import jax.numpy as jnp
from jax.experimental import pallas as pl
from jax.experimental.pallas import tpu as pltpu

B = 2048
D = 2048
K = 64
T = 32
NC = B // T
MU = 0.9
RHO_A = 0.99
DELTA = 1.0
ALPHA_P = 1.0
ALPHA_R = 0.5

_HI = jax.lax.Precision.DEFAULT


def _dot11(a, b):
    return jax.lax.dot_general(a, b, (((1,), (1,)), ((), ())),
                               precision=_HI, preferred_element_type=jnp.float32)


def _dot00(a, b):
    return jax.lax.dot_general(a, b, (((0,), (0,)), ((), ())),
                               precision=_HI, preferred_element_type=jnp.float32)


def _vault_kernel(g_sref, init_sref, z_ref, ga_ref, proto_ref, ma_ref,
                  out_ref, P_scr):
    lane = jax.lax.broadcasted_iota(jnp.int32, (1, K), 1)
    sub = jax.lax.broadcasted_iota(jnp.int32, (T, 1), 0)
    eyeK = (jax.lax.broadcasted_iota(jnp.int32, (K, K), 0)
            == jax.lax.broadcasted_iota(jnp.int32, (K, K), 1))
    laneT = jax.lax.broadcasted_iota(jnp.int32, (1, T), 1)

    P_scr[...] = proto_ref[...]

    P0 = P_scr[...]
    PP0 = _dot11(P0, P0)
    pn_init = jnp.sum(jnp.where(eyeK, PP0, 0.0), axis=0, keepdims=True)

    def chunk_body(c, pn_in):
        Zc = z_ref[pl.ds(c * T, T), :]
        P = P_scr[...]
        H = _dot11(Zc, P)
        G = _dot11(Zc, Zc)
        zn_col = jnp.sum(Zc * Zc, axis=1, keepdims=True)

        def step(t, carry):
            row, Scur, a_row, C, pn = carry
            tn = t + 1
            rnext = jnp.sum(jnp.where(sub == tn, Scur, 0.0),
                            axis=0, keepdims=True)

            def active(cr):
                row, Scur, a_row, C, pn = cr
                d2 = pn - 2.0 * row
                m = jnp.min(d2, axis=1, keepdims=True)
                idxv = jnp.where(d2 == m, lane, K)
                kmin = jnp.min(idxv, axis=1, keepdims=True)
                khr = jnp.where(lane == kmin, 1.0, 0.0)
                znt = jnp.sum(jnp.where(sub == t, zn_col, 0.0),
                              axis=0, keepdims=True)
                Gcol = jnp.sum(jnp.where(laneT == t, G, 0.0),
                               axis=1, keepdims=True)
                gsc = jnp.sum(jnp.where(sub == tn, Gcol, 0.0),
                              axis=0, keepdims=True)
                tsc = jnp.where(sub == t, 1.0, 0.0)
                pn_upd = (MU * MU) * pn + (1.0 - MU) ** 2 * znt \
                    + 2.0 * MU * (1.0 - MU) * row
                pn = pn + khr * (pn_upd - pn)
                row2 = rnext + khr * ((MU - 1.0) * rnext + (1.0 - MU) * gsc)
                Scur = Scur + khr * ((MU - 1.0) * Scur + (1.0 - MU) * Gcol)
                a_row = a_row * (1.0 - (1.0 - MU) * khr)
                C = C * (1.0 - (1.0 - MU) * khr) + (1.0 - MU) * (tsc * khr)
                return row2, Scur, a_row, C, pn

            def inactive(cr):
                _, Scur, a_row, C, pn = cr
                return rnext, Scur, a_row, C, pn

            return jax.lax.cond(g_sref[c * T + t] > 0, active, inactive,
                                carry)

        carry0 = (H[0:1, :], H, jnp.ones((1, K), jnp.float32),
                  jnp.zeros((T, K), jnp.float32), pn_in)
        _, _, a_row, C, pn_out = jax.lax.fori_loop(0, T, step, carry0,
                                                   unroll=2)
        diagA = jnp.where(eyeK, a_row, 0.0)
        P_scr[...] = jnp.dot(diagA, P, precision=_HI,
                             preferred_element_type=jnp.float32) \
            + _dot00(C, Zc)
        return pn_out

    jax.lax.fori_loop(0, NC, chunk_body, pn_init)

    ga_row = ga_ref[...]
    Zfull = z_ref[...]
    anom = jax.lax.dot_general(ga_row, Zfull, (((1,), (0,)), ((), ())),
                               precision=_HI,
                               preferred_element_type=jnp.float32)
    cnt = jnp.sum(ga_row)
    z_a = anom / jnp.maximum(cnt, 1.0)
    ma_in = ma_ref[...]
    fi = jnp.where(init_sref[0] > 0, 1.0, 0.0)
    cf = jnp.where(cnt > 0.0, 1.0, 0.0)
    ma_upd = fi * (RHO_A * ma_in + (1.0 - RHO_A) * z_a) + (1.0 - fi) * z_a
    new_ma = cf * ma_upd + (1.0 - cf) * ma_in
    man2 = jnp.sum(new_ma * new_ma)
    new_init = jnp.maximum(fi, cf)

    Pf = P_scr[...]
    PG = _dot11(Pf, Pf)
    pn_row = jnp.sum(jnp.where(eyeK, PG, 0.0), axis=0, keepdims=True)
    laneTK = jax.lax.broadcasted_iota(jnp.int32, (T, K), 1)

    def loss_body(c, accs):
        pull_acc, push_acc = accs
        Zc = z_ref[pl.ds(c * T, T), :]
        zn_col = jnp.sum(Zc * Zc, axis=1, keepdims=True)
        H2 = _dot11(Zc, Pf)
        d2 = zn_col + pn_row - 2.0 * H2
        dcur = d2
        ehot = jnp.zeros((T, K), jnp.float32)
        for _ in range(3):
            m = jnp.min(dcur, axis=1, keepdims=True)
            idxv = jnp.where(dcur == m, laneTK, K)
            kmin = jnp.min(idxv, axis=1, keepdims=True)
            hot = (laneTK == kmin).astype(jnp.float32)
            ehot = ehot + hot
            dcur = jnp.where(hot > 0.0, jnp.float32(3e38), dcur)
        sumHtop = jnp.sum(ehot * H2, axis=1, keepdims=True)
        ePG = jnp.dot(ehot, PG, precision=_HI,
                      preferred_element_type=jnp.float32)
        quad = jnp.sum(ehot * ePG, axis=1, keepdims=True)
        pull_acc = pull_acc + jnp.sum(
            zn_col - (2.0 / 3.0) * sumHtop + (1.0 / 9.0) * quad)
        zma = _dot11(Zc, new_ma)
        da = jnp.sqrt(jnp.maximum(zn_col - 2.0 * zma + man2, 0.0))
        push_acc = push_acc + jnp.sum(jnp.maximum(DELTA - da, 0.0))
        return pull_acc, push_acc

    pull_acc, push_acc = jax.lax.fori_loop(
        0, NC, loss_body, (jnp.float32(0.0), jnp.float32(0.0)))

    l_pull = pull_acc / jnp.float32(B * D)
    l_push = new_init * (push_acc / jnp.float32(B))
    out_ref[0] = ALPHA_P * l_pull + ALPHA_R * l_push


@jax.jit
def kernel(z_t, g_t, prototypes, proto_age, m_a, m_a_init):
    del proto_age
    g_i32 = g_t.astype(jnp.int32)
    init_i32 = m_a_init.astype(jnp.int32).reshape((1,))
    ga = (g_t == 0).astype(jnp.float32).reshape(1, B)
    ma2d = m_a.reshape(1, D)

    grid_spec = pltpu.PrefetchScalarGridSpec(
        num_scalar_prefetch=2,
        grid=(1,),
        in_specs=[
            pl.BlockSpec((B, D), lambda i, *_: (0, 0)),
            pl.BlockSpec((1, B), lambda i, *_: (0, 0)),
            pl.BlockSpec((K, D), lambda i, *_: (0, 0)),
            pl.BlockSpec((1, D), lambda i, *_: (0, 0)),
        ],
        out_specs=pl.BlockSpec(memory_space=pltpu.SMEM),
        scratch_shapes=[
            pltpu.VMEM((K, D), jnp.float32),
        ],
    )
    out = pl.pallas_call(
        _vault_kernel,
        grid_spec=grid_spec,
        out_shape=jax.ShapeDtypeStruct((1,), jnp.float32),
    )(g_i32, init_i32, z_t, ga, prototypes, ma2d)
    return out.reshape(())


# speedup vs baseline: 1.0613x; 1.0613x over previous
import jax
import jax.numpy as jnp
from jax.experimental import pallas as pl
from jax.experimental.pallas import tpu as pltpu

B = 2048
D = 2048
K = 64
T = 128
NC = B // T
MU = 0.9
RHO_A = 0.99
DELTA = 1.0
ALPHA_P = 1.0
ALPHA_R = 0.5

_HI = jax.lax.Precision.DEFAULT


def _dot11(a, b):
    return jax.lax.dot_general(a, b, (((1,), (1,)), ((), ())),
                               precision=_HI, preferred_element_type=jnp.float32)


def _dot00(a, b):
    return jax.lax.dot_general(a, b, (((0,), (0,)), ((), ())),
                               precision=_HI, preferred_element_type=jnp.float32)


def _vault_kernel(g_sref, init_sref, z_ref, ga_ref, proto_ref, ma_ref,
                  out_ref, P_scr):
    lane = jax.lax.broadcasted_iota(jnp.int32, (1, K), 1)
    sub = jax.lax.broadcasted_iota(jnp.int32, (T, 1), 0)
    eyeK = (jax.lax.broadcasted_iota(jnp.int32, (K, K), 0)
            == jax.lax.broadcasted_iota(jnp.int32, (K, K), 1))
    laneT = jax.lax.broadcasted_iota(jnp.int32, (1, T), 1)

    P_scr[...] = proto_ref[...]

    P0 = P_scr[...]
    PP0 = _dot11(P0, P0)
    pn_init = jnp.sum(jnp.where(eyeK, PP0, 0.0), axis=0, keepdims=True)

    def chunk_body(c, pn_in):
        Zc = z_ref[pl.ds(c * T, T), :]
        P = P_scr[...]
        H = _dot11(Zc, P)
        G = _dot11(Zc, Zc)
        zn_col = jnp.sum(Zc * Zc, axis=1, keepdims=True)

        def step(t, carry):
            row, Scur, a_row, C, pn = carry
            tn = t + 1
            rnext = jnp.sum(jnp.where(sub == tn, Scur, 0.0),
                            axis=0, keepdims=True)

            def active(cr):
                row, Scur, a_row, C, pn = cr
                d2 = pn - 2.0 * row
                m = jnp.min(d2, axis=1, keepdims=True)
                idxv = jnp.where(d2 == m, lane, K)
                kmin = jnp.min(idxv, axis=1, keepdims=True)
                khr = jnp.where(lane == kmin, 1.0, 0.0)
                znt = jnp.sum(jnp.where(sub == t, zn_col, 0.0),
                              axis=0, keepdims=True)
                Gcol = jnp.sum(jnp.where(laneT == t, G, 0.0),
                               axis=1, keepdims=True)
                gsc = jnp.sum(jnp.where(sub == tn, Gcol, 0.0),
                              axis=0, keepdims=True)
                tsc = jnp.where(sub == t, 1.0, 0.0)
                pn_upd = (MU * MU) * pn + (1.0 - MU) ** 2 * znt \
                    + 2.0 * MU * (1.0 - MU) * row
                pn = pn + khr * (pn_upd - pn)
                row2 = rnext + khr * ((MU - 1.0) * rnext + (1.0 - MU) * gsc)
                Scur = Scur + khr * ((MU - 1.0) * Scur + (1.0 - MU) * Gcol)
                a_row = a_row * (1.0 - (1.0 - MU) * khr)
                C = C * (1.0 - (1.0 - MU) * khr) + (1.0 - MU) * (tsc * khr)
                return row2, Scur, a_row, C, pn

            def inactive(cr):
                _, Scur, a_row, C, pn = cr
                return rnext, Scur, a_row, C, pn

            return jax.lax.cond(g_sref[c * T + t] > 0, active, inactive,
                                carry)

        carry0 = (H[0:1, :], H, jnp.ones((1, K), jnp.float32),
                  jnp.zeros((T, K), jnp.float32), pn_in)
        _, _, a_row, C, pn_out = jax.lax.fori_loop(0, T, step, carry0,
                                                   unroll=2)
        diagA = jnp.where(eyeK, a_row, 0.0)
        P_scr[...] = jnp.dot(diagA, P, precision=_HI,
                             preferred_element_type=jnp.float32) \
            + _dot00(C, Zc)
        return pn_out

    jax.lax.fori_loop(0, NC, chunk_body, pn_init)

    ga_row = ga_ref[...]
    Zfull = z_ref[...]
    anom = jax.lax.dot_general(ga_row, Zfull, (((1,), (0,)), ((), ())),
                               precision=_HI,
                               preferred_element_type=jnp.float32)
    cnt = jnp.sum(ga_row)
    z_a = anom / jnp.maximum(cnt, 1.0)
    ma_in = ma_ref[...]
    fi = jnp.where(init_sref[0] > 0, 1.0, 0.0)
    cf = jnp.where(cnt > 0.0, 1.0, 0.0)
    ma_upd = fi * (RHO_A * ma_in + (1.0 - RHO_A) * z_a) + (1.0 - fi) * z_a
    new_ma = cf * ma_upd + (1.0 - cf) * ma_in
    man2 = jnp.sum(new_ma * new_ma)
    new_init = jnp.maximum(fi, cf)

    Pf = P_scr[...]
    PG = _dot11(Pf, Pf)
    pn_row = jnp.sum(jnp.where(eyeK, PG, 0.0), axis=0, keepdims=True)
    laneTK = jax.lax.broadcasted_iota(jnp.int32, (T, K), 1)

    def loss_body(c, accs):
        pull_acc, push_acc = accs
        Zc = z_ref[pl.ds(c * T, T), :]
        zn_col = jnp.sum(Zc * Zc, axis=1, keepdims=True)
        H2 = _dot11(Zc, Pf)
        d2 = zn_col + pn_row - 2.0 * H2
        dcur = d2
        ehot = jnp.zeros((T, K), jnp.float32)
        for _ in range(3):
            m = jnp.min(dcur, axis=1, keepdims=True)
            idxv = jnp.where(dcur == m, laneTK, K)
            kmin = jnp.min(idxv, axis=1, keepdims=True)
            hot = (laneTK == kmin).astype(jnp.float32)
            ehot = ehot + hot
            dcur = jnp.where(hot > 0.0, jnp.float32(3e38), dcur)
        sumHtop = jnp.sum(ehot * H2, axis=1, keepdims=True)
        ePG = jnp.dot(ehot, PG, precision=_HI,
                      preferred_element_type=jnp.float32)
        quad = jnp.sum(ehot * ePG, axis=1, keepdims=True)
        pull_acc = pull_acc + jnp.sum(
            zn_col - (2.0 / 3.0) * sumHtop + (1.0 / 9.0) * quad)
        zma = _dot11(Zc, new_ma)
        da = jnp.sqrt(jnp.maximum(zn_col - 2.0 * zma + man2, 0.0))
        push_acc = push_acc + jnp.sum(jnp.maximum(DELTA - da, 0.0))
        return pull_acc, push_acc

    pull_acc, push_acc = jax.lax.fori_loop(
        0, NC, loss_body, (jnp.float32(0.0), jnp.float32(0.0)))

    l_pull = pull_acc / jnp.float32(B * D)
    l_push = new_init * (push_acc / jnp.float32(B))
    out_ref[0] = ALPHA_P * l_pull + ALPHA_R * l_push


@jax.jit
def kernel(z_t, g_t, prototypes, proto_age, m_a, m_a_init):
    del proto_age
    g_i32 = g_t.astype(jnp.int32)
    init_i32 = m_a_init.astype(jnp.int32).reshape((1,))
    ga = (g_t == 0).astype(jnp.float32).reshape(1, B)
    ma2d = m_a.reshape(1, D)

    grid_spec = pltpu.PrefetchScalarGridSpec(
        num_scalar_prefetch=2,
        grid=(1,),
        in_specs=[
            pl.BlockSpec((B, D), lambda i, *_: (0, 0)),
            pl.BlockSpec((1, B), lambda i, *_: (0, 0)),
            pl.BlockSpec((K, D), lambda i, *_: (0, 0)),
            pl.BlockSpec((1, D), lambda i, *_: (0, 0)),
        ],
        out_specs=pl.BlockSpec(memory_space=pltpu.SMEM),
        scratch_shapes=[
            pltpu.VMEM((K, D), jnp.float32),
        ],
    )
    out = pl.pallas_call(
        _vault_kernel,
        grid_spec=grid_spec,
        out_shape=jax.ShapeDtypeStruct((1,), jnp.float32),
    )(g_i32, init_i32, z_t, ga, prototypes, ma2d)
    return out.reshape(())


# speedup vs baseline: 1.2308x; 1.1596x over previous
import jax
import jax.numpy as jnp
from jax.experimental import pallas as pl
from jax.experimental.pallas import tpu as pltpu

B = 2048
D = 2048
K = 64
T = 64
NC = B // T
MU = 0.9
RHO_A = 0.99
DELTA = 1.0
ALPHA_P = 1.0
ALPHA_R = 0.5

_HI = jax.lax.Precision.DEFAULT


def _dot11(a, b):
    return jax.lax.dot_general(a, b, (((1,), (1,)), ((), ())),
                               precision=_HI, preferred_element_type=jnp.float32)


def _dot00(a, b):
    return jax.lax.dot_general(a, b, (((0,), (0,)), ((), ())),
                               precision=_HI, preferred_element_type=jnp.float32)


def _vault_kernel(g_sref, init_sref, z_ref, ga_ref, proto_ref, ma_ref,
                  out_ref, P_scr):
    lane = jax.lax.broadcasted_iota(jnp.int32, (1, K), 1)
    sub = jax.lax.broadcasted_iota(jnp.int32, (T, 1), 0)
    eyeK = (jax.lax.broadcasted_iota(jnp.int32, (K, K), 0)
            == jax.lax.broadcasted_iota(jnp.int32, (K, K), 1))
    laneT = jax.lax.broadcasted_iota(jnp.int32, (1, T), 1)

    P_scr[...] = proto_ref[...]

    P0 = P_scr[...]
    PP0 = _dot11(P0, P0)
    pn_init = jnp.sum(jnp.where(eyeK, PP0, 0.0), axis=0, keepdims=True)

    def chunk_body(c, pn_in):
        Zc = z_ref[pl.ds(c * T, T), :]
        P = P_scr[...]
        H = _dot11(Zc, P)
        G = _dot11(Zc, Zc)
        zn_col = jnp.sum(Zc * Zc, axis=1, keepdims=True)

        def step(t, carry):
            row, Scur, a_row, C, pn = carry
            tn = t + 1
            rnext = jnp.sum(jnp.where(sub == tn, Scur, 0.0),
                            axis=0, keepdims=True)

            def active(cr):
                row, Scur, a_row, C, pn = cr
                d2 = pn - 2.0 * row
                m = jnp.min(d2, axis=1, keepdims=True)
                idxv = jnp.where(d2 == m, lane, K)
                kmin = jnp.min(idxv, axis=1, keepdims=True)
                khr = jnp.where(lane == kmin, 1.0, 0.0)
                znt = jnp.sum(jnp.where(sub == t, zn_col, 0.0),
                              axis=0, keepdims=True)
                Gcol = jnp.sum(jnp.where(laneT == t, G, 0.0),
                               axis=1, keepdims=True)
                gsc = jnp.sum(jnp.where(sub == tn, Gcol, 0.0),
                              axis=0, keepdims=True)
                tsc = jnp.where(sub == t, 1.0, 0.0)
                pn_upd = (MU * MU) * pn + (1.0 - MU) ** 2 * znt \
                    + 2.0 * MU * (1.0 - MU) * row
                pn = pn + khr * (pn_upd - pn)
                row2 = rnext + khr * ((MU - 1.0) * rnext + (1.0 - MU) * gsc)
                Scur = Scur + khr * ((MU - 1.0) * Scur + (1.0 - MU) * Gcol)
                a_row = a_row * (1.0 - (1.0 - MU) * khr)
                C = C * (1.0 - (1.0 - MU) * khr) + (1.0 - MU) * (tsc * khr)
                return row2, Scur, a_row, C, pn

            def inactive(cr):
                _, Scur, a_row, C, pn = cr
                return rnext, Scur, a_row, C, pn

            return jax.lax.cond(g_sref[c * T + t] > 0, active, inactive,
                                carry)

        carry0 = (H[0:1, :], H, jnp.ones((1, K), jnp.float32),
                  jnp.zeros((T, K), jnp.float32), pn_in)
        _, _, a_row, C, pn_out = jax.lax.fori_loop(0, T, step, carry0,
                                                   unroll=2)
        diagA = jnp.where(eyeK, a_row, 0.0)
        P_scr[...] = jnp.dot(diagA, P, precision=_HI,
                             preferred_element_type=jnp.float32) \
            + _dot00(C, Zc)
        return pn_out

    jax.lax.fori_loop(0, NC, chunk_body, pn_init)

    ga_row = ga_ref[...]
    Zfull = z_ref[...]
    anom = jax.lax.dot_general(ga_row, Zfull, (((1,), (0,)), ((), ())),
                               precision=_HI,
                               preferred_element_type=jnp.float32)
    cnt = jnp.sum(ga_row)
    z_a = anom / jnp.maximum(cnt, 1.0)
    ma_in = ma_ref[...]
    fi = jnp.where(init_sref[0] > 0, 1.0, 0.0)
    cf = jnp.where(cnt > 0.0, 1.0, 0.0)
    ma_upd = fi * (RHO_A * ma_in + (1.0 - RHO_A) * z_a) + (1.0 - fi) * z_a
    new_ma = cf * ma_upd + (1.0 - cf) * ma_in
    man2 = jnp.sum(new_ma * new_ma)
    new_init = jnp.maximum(fi, cf)

    TL = 256
    NL = B // TL
    Pf = P_scr[...]
    PG = _dot11(Pf, Pf)
    pn_row = jnp.sum(jnp.where(eyeK, PG, 0.0), axis=0, keepdims=True)
    laneTK = jax.lax.broadcasted_iota(jnp.int32, (TL, K), 1)

    def loss_body(c, accs):
        pull_acc, push_acc = accs
        Zc = z_ref[pl.ds(c * TL, TL), :]
        zn_col = jnp.sum(Zc * Zc, axis=1, keepdims=True)
        H2 = _dot11(Zc, Pf)
        d2 = zn_col + pn_row - 2.0 * H2
        dcur = d2
        ehot = jnp.zeros((TL, K), jnp.float32)
        for _ in range(3):
            m = jnp.min(dcur, axis=1, keepdims=True)
            idxv = jnp.where(dcur == m, laneTK, K)
            kmin = jnp.min(idxv, axis=1, keepdims=True)
            hot = (laneTK == kmin).astype(jnp.float32)
            ehot = ehot + hot
            dcur = jnp.where(hot > 0.0, jnp.float32(3e38), dcur)
        sumHtop = jnp.sum(ehot * H2, axis=1, keepdims=True)
        ePG = jnp.dot(ehot, PG, precision=_HI,
                      preferred_element_type=jnp.float32)
        quad = jnp.sum(ehot * ePG, axis=1, keepdims=True)
        pull_acc = pull_acc + jnp.sum(
            zn_col - (2.0 / 3.0) * sumHtop + (1.0 / 9.0) * quad)
        zma = _dot11(Zc, new_ma)
        da = jnp.sqrt(jnp.maximum(zn_col - 2.0 * zma + man2, 0.0))
        push_acc = push_acc + jnp.sum(jnp.maximum(DELTA - da, 0.0))
        return pull_acc, push_acc

    pull_acc, push_acc = jax.lax.fori_loop(
        0, NL, loss_body, (jnp.float32(0.0), jnp.float32(0.0)))

    l_pull = pull_acc / jnp.float32(B * D)
    l_push = new_init * (push_acc / jnp.float32(B))
    out_ref[0] = ALPHA_P * l_pull + ALPHA_R * l_push


@jax.jit
def kernel(z_t, g_t, prototypes, proto_age, m_a, m_a_init):
    del proto_age
    g_i32 = g_t.astype(jnp.int32)
    init_i32 = m_a_init.astype(jnp.int32).reshape((1,))
    ga = (g_t == 0).astype(jnp.float32).reshape(1, B)
    ma2d = m_a.reshape(1, D)

    grid_spec = pltpu.PrefetchScalarGridSpec(
        num_scalar_prefetch=2,
        grid=(1,),
        in_specs=[
            pl.BlockSpec((B, D), lambda i, *_: (0, 0)),
            pl.BlockSpec((1, B), lambda i, *_: (0, 0)),
            pl.BlockSpec((K, D), lambda i, *_: (0, 0)),
            pl.BlockSpec((1, D), lambda i, *_: (0, 0)),
        ],
        out_specs=pl.BlockSpec(memory_space=pltpu.SMEM),
        scratch_shapes=[
            pltpu.VMEM((K, D), jnp.float32),
        ],
    )
    out = pl.pallas_call(
        _vault_kernel,
        grid_spec=grid_spec,
        out_shape=jax.ShapeDtypeStruct((1,), jnp.float32),
    )(g_i32, init_i32, z_t, ga, prototypes, ma2d)
    return out.reshape(())


# speedup vs baseline: 1.2445x; 1.0112x over previous
import jax
import jax.numpy as jnp
from jax.experimental import pallas as pl
from jax.experimental.pallas import tpu as pltpu

B = 2048
D = 2048
K = 64
T = 64
NC = B // T
MU = 0.9
RHO_A = 0.99
DELTA = 1.0
ALPHA_P = 1.0
ALPHA_R = 0.5

_HI = jax.lax.Precision.DEFAULT


def _dot11(a, b):
    return jax.lax.dot_general(a, b, (((1,), (1,)), ((), ())),
                               precision=_HI, preferred_element_type=jnp.float32)


def _dot00(a, b):
    return jax.lax.dot_general(a, b, (((0,), (0,)), ((), ())),
                               precision=_HI, preferred_element_type=jnp.float32)


def _vault_kernel(g_sref, init_sref, z_ref, ga_ref, proto_ref, ma_ref,
                  out_ref, P_scr, khr_scr):
    lane = jax.lax.broadcasted_iota(jnp.int32, (1, K), 1)
    sub = jax.lax.broadcasted_iota(jnp.int32, (T, 1), 0)
    eyeK = (jax.lax.broadcasted_iota(jnp.int32, (K, K), 0)
            == jax.lax.broadcasted_iota(jnp.int32, (K, K), 1))
    laneT = jax.lax.broadcasted_iota(jnp.int32, (1, T), 1)

    P_scr[...] = proto_ref[...]

    P0 = P_scr[...]
    PP0 = _dot11(P0, P0)
    pn_init = jnp.sum(jnp.where(eyeK, PP0, 0.0), axis=0, keepdims=True)

    trilT = jnp.where(
        jax.lax.broadcasted_iota(jnp.int32, (T, T), 0)
        >= jax.lax.broadcasted_iota(jnp.int32, (T, T), 1), 1.0, 0.0)
    LN_MU = float(__import__("math").log(MU))

    def chunk_body(c, pn_in):
        Zc = z_ref[pl.ds(c * T, T), :]
        P = P_scr[...]
        H = _dot11(Zc, P)
        G = _dot11(Zc, Zc)
        zn_col = jnp.sum(Zc * Zc, axis=1, keepdims=True)
        khr_scr[...] = jnp.zeros((T, K), jnp.float32)

        def step(t, carry):
            row, Scur, pn = carry
            tn = t + 1
            rnext = jnp.sum(jnp.where(sub == tn, Scur, 0.0),
                            axis=0, keepdims=True)

            def active(cr):
                row, Scur, pn = cr
                d2 = pn - 2.0 * row
                m = jnp.min(d2, axis=1, keepdims=True)
                idxv = jnp.where(d2 == m, lane, K)
                kmin = jnp.min(idxv, axis=1, keepdims=True)
                khr = jnp.where(lane == kmin, 1.0, 0.0)
                khr_scr[pl.ds(t, 1), :] = khr
                znt = jnp.sum(jnp.where(sub == t, zn_col, 0.0),
                              axis=0, keepdims=True)
                Gcol = jnp.sum(jnp.where(laneT == t, G, 0.0),
                               axis=1, keepdims=True)
                gsc = jnp.sum(jnp.where(sub == tn, Gcol, 0.0),
                              axis=0, keepdims=True)
                pn_upd = (MU * MU) * pn + (1.0 - MU) ** 2 * znt \
                    + 2.0 * MU * (1.0 - MU) * row
                pn = pn + khr * (pn_upd - pn)
                row2 = rnext + khr * ((MU - 1.0) * rnext + (1.0 - MU) * gsc)
                Scur = Scur + khr * ((MU - 1.0) * Scur + (1.0 - MU) * Gcol)
                return row2, Scur, pn

            def inactive(cr):
                _, Scur, pn = cr
                return rnext, Scur, pn

            return jax.lax.cond(g_sref[c * T + t] > 0, active, inactive,
                                carry)

        carry0 = (H[0:1, :], H, pn_in)
        _, _, pn_out = jax.lax.fori_loop(0, T, step, carry0, unroll=2)
        Khr = khr_scr[...]
        cum = jnp.dot(trilT, Khr, precision=_HI,
                      preferred_element_type=jnp.float32)
        total = jnp.sum(Khr, axis=0, keepdims=True)
        C = (1.0 - MU) * Khr * jnp.exp((total - cum) * LN_MU)
        a_row = jnp.exp(total * LN_MU)
        diagA = jnp.where(eyeK, a_row, 0.0)
        P_scr[...] = jnp.dot(diagA, P, precision=_HI,
                             preferred_element_type=jnp.float32) \
            + _dot00(C, Zc)
        return pn_out

    jax.lax.fori_loop(0, NC, chunk_body, pn_init)

    ga_row = ga_ref[...]
    Zfull = z_ref[...]
    anom = jax.lax.dot_general(ga_row, Zfull, (((1,), (0,)), ((), ())),
                               precision=_HI,
                               preferred_element_type=jnp.float32)
    cnt = jnp.sum(ga_row)
    z_a = anom / jnp.maximum(cnt, 1.0)
    ma_in = ma_ref[...]
    fi = jnp.where(init_sref[0] > 0, 1.0, 0.0)
    cf = jnp.where(cnt > 0.0, 1.0, 0.0)
    ma_upd = fi * (RHO_A * ma_in + (1.0 - RHO_A) * z_a) + (1.0 - fi) * z_a
    new_ma = cf * ma_upd + (1.0 - cf) * ma_in
    man2 = jnp.sum(new_ma * new_ma)
    new_init = jnp.maximum(fi, cf)

    TL = 256
    NL = B // TL
    Pf = P_scr[...]
    PG = _dot11(Pf, Pf)
    pn_row = jnp.sum(jnp.where(eyeK, PG, 0.0), axis=0, keepdims=True)
    laneTK = jax.lax.broadcasted_iota(jnp.int32, (TL, K), 1)

    def loss_body(c, accs):
        pull_acc, push_acc = accs
        Zc = z_ref[pl.ds(c * TL, TL), :]
        zn_col = jnp.sum(Zc * Zc, axis=1, keepdims=True)
        H2 = _dot11(Zc, Pf)
        d2 = zn_col + pn_row - 2.0 * H2
        dcur = d2
        ehot = jnp.zeros((TL, K), jnp.float32)
        for _ in range(3):
            m = jnp.min(dcur, axis=1, keepdims=True)
            idxv = jnp.where(dcur == m, laneTK, K)
            kmin = jnp.min(idxv, axis=1, keepdims=True)
            hot = (laneTK == kmin).astype(jnp.float32)
            ehot = ehot + hot
            dcur = jnp.where(hot > 0.0, jnp.float32(3e38), dcur)
        sumHtop = jnp.sum(ehot * H2, axis=1, keepdims=True)
        ePG = jnp.dot(ehot, PG, precision=_HI,
                      preferred_element_type=jnp.float32)
        quad = jnp.sum(ehot * ePG, axis=1, keepdims=True)
        pull_acc = pull_acc + jnp.sum(
            zn_col - (2.0 / 3.0) * sumHtop + (1.0 / 9.0) * quad)
        zma = _dot11(Zc, new_ma)
        da = jnp.sqrt(jnp.maximum(zn_col - 2.0 * zma + man2, 0.0))
        push_acc = push_acc + jnp.sum(jnp.maximum(DELTA - da, 0.0))
        return pull_acc, push_acc

    pull_acc, push_acc = jax.lax.fori_loop(
        0, NL, loss_body, (jnp.float32(0.0), jnp.float32(0.0)))

    l_pull = pull_acc / jnp.float32(B * D)
    l_push = new_init * (push_acc / jnp.float32(B))
    out_ref[0] = ALPHA_P * l_pull + ALPHA_R * l_push


@jax.jit
def kernel(z_t, g_t, prototypes, proto_age, m_a, m_a_init):
    del proto_age
    g_i32 = g_t.astype(jnp.int32)
    init_i32 = m_a_init.astype(jnp.int32).reshape((1,))
    ga = (g_t == 0).astype(jnp.float32).reshape(1, B)
    ma2d = m_a.reshape(1, D)

    grid_spec = pltpu.PrefetchScalarGridSpec(
        num_scalar_prefetch=2,
        grid=(1,),
        in_specs=[
            pl.BlockSpec((B, D), lambda i, *_: (0, 0)),
            pl.BlockSpec((1, B), lambda i, *_: (0, 0)),
            pl.BlockSpec((K, D), lambda i, *_: (0, 0)),
            pl.BlockSpec((1, D), lambda i, *_: (0, 0)),
        ],
        out_specs=pl.BlockSpec(memory_space=pltpu.SMEM),
        scratch_shapes=[
            pltpu.VMEM((K, D), jnp.float32),
            pltpu.VMEM((T, K), jnp.float32),
        ],
    )
    out = pl.pallas_call(
        _vault_kernel,
        grid_spec=grid_spec,
        out_shape=jax.ShapeDtypeStruct((1,), jnp.float32),
    )(g_i32, init_i32, z_t, ga, prototypes, ma2d)
    return out.reshape(())


# speedup vs baseline: 1.2484x; 1.0031x over previous
import jax
import jax.numpy as jnp
from jax.experimental import pallas as pl
from jax.experimental.pallas import tpu as pltpu

B = 2048
D = 2048
K = 64
T = 64
NC = B // T
MU = 0.9
RHO_A = 0.99
DELTA = 1.0
ALPHA_P = 1.0
ALPHA_R = 0.5

_HI = jax.lax.Precision.DEFAULT


def _dot11(a, b):
    return jax.lax.dot_general(a, b, (((1,), (1,)), ((), ())),
                               precision=_HI, preferred_element_type=jnp.float32)


def _dot00(a, b):
    return jax.lax.dot_general(a, b, (((0,), (0,)), ((), ())),
                               precision=_HI, preferred_element_type=jnp.float32)


def _vault_kernel(g_sref, init_sref, z_ref, ga_ref, proto_ref, ma_ref,
                  out_ref, P_scr, khr_scr):
    lane = jax.lax.broadcasted_iota(jnp.int32, (1, K), 1)
    sub = jax.lax.broadcasted_iota(jnp.int32, (T, 1), 0)
    eyeK = (jax.lax.broadcasted_iota(jnp.int32, (K, K), 0)
            == jax.lax.broadcasted_iota(jnp.int32, (K, K), 1))
    laneT = jax.lax.broadcasted_iota(jnp.int32, (1, T), 1)

    P_scr[...] = proto_ref[...]

    P0 = P_scr[...]
    PP0 = _dot11(P0, P0)
    pn_init = jnp.sum(jnp.where(eyeK, PP0, 0.0), axis=0, keepdims=True)

    trilT = jnp.where(
        jax.lax.broadcasted_iota(jnp.int32, (T, T), 0)
        >= jax.lax.broadcasted_iota(jnp.int32, (T, T), 1), 1.0, 0.0)
    LN_MU = float(__import__("math").log(MU))

    def chunk_body(c, pn_in):
        Zc = z_ref[pl.ds(c * T, T), :]
        P = P_scr[...]
        H = _dot11(Zc, P)
        G = _dot11(Zc, Zc)
        zn_col = jnp.sum(Zc * Zc, axis=1, keepdims=True)
        khr_scr[...] = jnp.zeros((T, K), jnp.float32)

        def step(t, carry):
            row, Scur, pn = carry
            tn = t + 1
            rnext = jnp.sum(jnp.where(sub == tn, Scur, 0.0),
                            axis=0, keepdims=True)

            def active(cr):
                row, Scur, pn = cr
                d2 = pn - 2.0 * row
                m = jnp.min(d2, axis=1, keepdims=True)
                idxv = jnp.where(d2 == m, lane, K)
                kmin = jnp.min(idxv, axis=1, keepdims=True)
                khr = jnp.where(lane == kmin, 1.0, 0.0)
                khr_scr[pl.ds(t, 1), :] = khr
                znt = jnp.sum(jnp.where(sub == t, zn_col, 0.0),
                              axis=0, keepdims=True)
                Gcol = jnp.sum(jnp.where(laneT == t, G, 0.0),
                               axis=1, keepdims=True)
                gsc = jnp.sum(jnp.where(sub == tn, Gcol, 0.0),
                              axis=0, keepdims=True)
                pn_upd = (MU * MU) * pn + (1.0 - MU) ** 2 * znt \
                    + 2.0 * MU * (1.0 - MU) * row
                pn = pn + khr * (pn_upd - pn)
                row2 = rnext + khr * ((MU - 1.0) * rnext + (1.0 - MU) * gsc)
                Scur = Scur + khr * ((MU - 1.0) * Scur + (1.0 - MU) * Gcol)
                return row2, Scur, pn

            def inactive(cr):
                _, Scur, pn = cr
                return rnext, Scur, pn

            return jax.lax.cond(g_sref[c * T + t] > 0, active, inactive,
                                carry)

        carry0 = (H[0:1, :], H, pn_in)
        _, _, pn_out = jax.lax.fori_loop(0, T, step, carry0, unroll=4)
        Khr = khr_scr[...]
        cum = jnp.dot(trilT, Khr, precision=_HI,
                      preferred_element_type=jnp.float32)
        total = jnp.sum(Khr, axis=0, keepdims=True)
        C = (1.0 - MU) * Khr * jnp.exp((total - cum) * LN_MU)
        a_row = jnp.exp(total * LN_MU)
        diagA = jnp.where(eyeK, a_row, 0.0)
        P_scr[...] = jnp.dot(diagA, P, precision=_HI,
                             preferred_element_type=jnp.float32) \
            + _dot00(C, Zc)
        return pn_out

    jax.lax.fori_loop(0, NC, chunk_body, pn_init)

    ga_row = ga_ref[...]
    Zfull = z_ref[...]
    anom = jax.lax.dot_general(ga_row, Zfull, (((1,), (0,)), ((), ())),
                               precision=_HI,
                               preferred_element_type=jnp.float32)
    cnt = jnp.sum(ga_row)
    z_a = anom / jnp.maximum(cnt, 1.0)
    ma_in = ma_ref[...]
    fi = jnp.where(init_sref[0] > 0, 1.0, 0.0)
    cf = jnp.where(cnt > 0.0, 1.0, 0.0)
    ma_upd = fi * (RHO_A * ma_in + (1.0 - RHO_A) * z_a) + (1.0 - fi) * z_a
    new_ma = cf * ma_upd + (1.0 - cf) * ma_in
    man2 = jnp.sum(new_ma * new_ma)
    new_init = jnp.maximum(fi, cf)

    TL = 256
    NL = B // TL
    Pf = P_scr[...]
    PG = _dot11(Pf, Pf)
    pn_row = jnp.sum(jnp.where(eyeK, PG, 0.0), axis=0, keepdims=True)
    laneTK = jax.lax.broadcasted_iota(jnp.int32, (TL, K), 1)

    def loss_body(c, accs):
        pull_acc, push_acc = accs
        Zc = z_ref[pl.ds(c * TL, TL), :]
        zn_col = jnp.sum(Zc * Zc, axis=1, keepdims=True)
        H2 = _dot11(Zc, Pf)
        d2 = zn_col + pn_row - 2.0 * H2
        dcur = d2
        ehot = jnp.zeros((TL, K), jnp.float32)
        for _ in range(3):
            m = jnp.min(dcur, axis=1, keepdims=True)
            idxv = jnp.where(dcur == m, laneTK, K)
            kmin = jnp.min(idxv, axis=1, keepdims=True)
            hot = (laneTK == kmin).astype(jnp.float32)
            ehot = ehot + hot
            dcur = jnp.where(hot > 0.0, jnp.float32(3e38), dcur)
        sumHtop = jnp.sum(ehot * H2, axis=1, keepdims=True)
        ePG = jnp.dot(ehot, PG, precision=_HI,
                      preferred_element_type=jnp.float32)
        quad = jnp.sum(ehot * ePG, axis=1, keepdims=True)
        pull_acc = pull_acc + jnp.sum(
            zn_col - (2.0 / 3.0) * sumHtop + (1.0 / 9.0) * quad)
        zma = _dot11(Zc, new_ma)
        da = jnp.sqrt(jnp.maximum(zn_col - 2.0 * zma + man2, 0.0))
        push_acc = push_acc + jnp.sum(jnp.maximum(DELTA - da, 0.0))
        return pull_acc, push_acc

    pull_acc, push_acc = jax.lax.fori_loop(
        0, NL, loss_body, (jnp.float32(0.0), jnp.float32(0.0)))

    l_pull = pull_acc / jnp.float32(B * D)
    l_push = new_init * (push_acc / jnp.float32(B))
    out_ref[0] = ALPHA_P * l_pull + ALPHA_R * l_push


@jax.jit
def kernel(z_t, g_t, prototypes, proto_age, m_a, m_a_init):
    del proto_age
    g_i32 = g_t.astype(jnp.int32)
    init_i32 = m_a_init.astype(jnp.int32).reshape((1,))
    ga = (g_t == 0).astype(jnp.float32).reshape(1, B)
    ma2d = m_a.reshape(1, D)

    grid_spec = pltpu.PrefetchScalarGridSpec(
        num_scalar_prefetch=2,
        grid=(1,),
        in_specs=[
            pl.BlockSpec((B, D), lambda i, *_: (0, 0)),
            pl.BlockSpec((1, B), lambda i, *_: (0, 0)),
            pl.BlockSpec((K, D), lambda i, *_: (0, 0)),
            pl.BlockSpec((1, D), lambda i, *_: (0, 0)),
        ],
        out_specs=pl.BlockSpec(memory_space=pltpu.SMEM),
        scratch_shapes=[
            pltpu.VMEM((K, D), jnp.float32),
            pltpu.VMEM((T, K), jnp.float32),
        ],
    )
    out = pl.pallas_call(
        _vault_kernel,
        grid_spec=grid_spec,
        out_shape=jax.ShapeDtypeStruct((1,), jnp.float32),
    )(g_i32, init_i32, z_t, ga, prototypes, ma2d)
    return out.reshape(())
